# baseline (speedup 1.0000x reference)
import functools

import jax
import jax.numpy as jnp
from jax import lax
from jax.experimental import pallas as pl
from jax.experimental.pallas import tpu as pltpu
from jax.experimental.pallas import tpu_sc as plsc

N = 10000
E = 320000
H = 128
DP = 144
C = 8

NC, NS = 2, 16
NW = NC * NS
NP = 10240
RPT = NP // NS

_TC_PARAMS = pltpu.CompilerParams(dimension_semantics=("parallel",))


def _embed_body(x_ref, w_ref, b_ref, out_ref):
    h = jnp.dot(x_ref[...], w_ref[...], preferred_element_type=jnp.float32)
    h = h + b_ref[...]
    ones = jnp.ones((h.shape[0], DP - H), jnp.float32)
    out_ref[...] = jnp.concatenate([h, ones], axis=1)


_embed = pl.pallas_call(
    _embed_body,
    grid=(10,),
    in_specs=[
        pl.BlockSpec((1000, H), lambda i: (i, 0)),
        pl.BlockSpec((H, H), lambda i: (0, 0)),
        pl.BlockSpec((1, H), lambda i: (0, 0)),
    ],
    out_specs=pl.BlockSpec((1000, DP), lambda i: (i, 0)),
    out_shape=jax.ShapeDtypeStruct((N, DP), jnp.float32),
    compiler_params=_TC_PARAMS,
)


def _layer1_body(a0_ref, a1_ref, hp_ref, wl_ref, wr_ref, b_ref, out_ref):
    a = a0_ref[...] + a1_ref[...]
    deg = a[:, H:H + 1]
    cinv = 1.0 / jnp.maximum(deg, 1.0)
    mean = a[:, :H] * cinv
    h0 = hp_ref[:, :H]
    out = (jnp.dot(mean, wl_ref[...], preferred_element_type=jnp.float32)
           + jnp.dot(h0, wr_ref[...], preferred_element_type=jnp.float32)
           + b_ref[...])
    out_ref[...] = jnp.maximum(out, 0.0)


_layer1 = pl.pallas_call(
    _layer1_body,
    grid=(10,),
    in_specs=[
        pl.BlockSpec((None, 1000, DP), lambda i: (0, i, 0)),
        pl.BlockSpec((None, 1000, DP), lambda i: (1, i, 0)),
        pl.BlockSpec((1000, DP), lambda i: (i, 0)),
        pl.BlockSpec((H, H), lambda i: (0, 0)),
        pl.BlockSpec((H, H), lambda i: (0, 0)),
        pl.BlockSpec((1, H), lambda i: (0, 0)),
    ],
    out_specs=pl.BlockSpec((1000, H), lambda i: (i, 0)),
    out_shape=jax.ShapeDtypeStruct((N, H), jnp.float32),
    compiler_params=_TC_PARAMS,
)


def _layer2_body(a0_ref, a1_ref, c0_ref, c1_ref, h1_ref, wl_ref, wr_ref,
                 b_ref, whs_ref, whd_ref, t_out):
    a = a0_ref[...] + a1_ref[...]
    deg = c0_ref[:, H:H + 1] + c1_ref[:, H:H + 1]
    cinv = 1.0 / jnp.maximum(deg, 1.0)
    mean = a * cinv
    h2 = (jnp.dot(mean, wl_ref[...], preferred_element_type=jnp.float32)
          + jnp.dot(h1_ref[...], wr_ref[...], preferred_element_type=jnp.float32)
          + b_ref[...])
    h2 = jnp.maximum(h2, 0.0)

    def pack(v):
        u = jax.lax.bitcast_convert_type(v, jnp.int32)
        r = (u + 0x7FFF + ((u >> 16) & 1)) & jnp.int32(-65536)
        hi = r[:, :64]
        lo = (r[:, 64:] >> 16) & 0xFFFF
        return jax.lax.bitcast_convert_type(hi | lo, jnp.float32)

    t_out[...] = jnp.concatenate(
        [pack(jnp.dot(h2, whs_ref[...], preferred_element_type=jnp.float32)),
         pack(jnp.dot(h2, whd_ref[...], preferred_element_type=jnp.float32))],
        axis=1)


_layer2 = pl.pallas_call(
    _layer2_body,
    grid=(10,),
    in_specs=[
        pl.BlockSpec((None, 1000, H), lambda i: (0, i, 0)),
        pl.BlockSpec((None, 1000, H), lambda i: (1, i, 0)),
        pl.BlockSpec((None, 1000, DP), lambda i: (0, i, 0)),
        pl.BlockSpec((None, 1000, DP), lambda i: (1, i, 0)),
        pl.BlockSpec((1000, H), lambda i: (i, 0)),
        pl.BlockSpec((H, H), lambda i: (0, 0)),
        pl.BlockSpec((H, H), lambda i: (0, 0)),
        pl.BlockSpec((1, H), lambda i: (0, 0)),
        pl.BlockSpec((H, H), lambda i: (0, 0)),
        pl.BlockSpec((H, H), lambda i: (0, 0)),
    ],
    out_specs=pl.BlockSpec((1000, H), lambda i: (i, 0)),
    out_shape=jax.ShapeDtypeStruct((N, H), jnp.float32),
    compiler_params=_TC_PARAMS,
)


def _classifier_body(g_ref, ea_ref, whe_ref, bh_ref, wo_ref, bo_ref,
                     out_ref):
    w = jax.lax.bitcast_convert_type(g_ref[...], jnp.int32)
    hi = jax.lax.bitcast_convert_type(w & jnp.int32(-65536), jnp.float32)
    lo = jax.lax.bitcast_convert_type(w << 16, jnp.float32)
    g = jnp.concatenate([hi[:, :64] + hi[:, 64:], lo[:, :64] + lo[:, 64:]],
                        axis=1)
    z = (g
         + jnp.dot(ea_ref[...], whe_ref[...], preferred_element_type=jnp.float32)
         + bh_ref[...])
    z = jnp.maximum(z, 0.0)
    out_ref[...] = jnp.dot(z, wo_ref[...], preferred_element_type=jnp.float32) + bo_ref[...]


_classifier = pl.pallas_call(
    _classifier_body,
    grid=(160,),
    in_specs=[
        pl.BlockSpec((2000, H), lambda i: (i, 0)),
        pl.BlockSpec((2000, 16), lambda i: (i, 0)),
        pl.BlockSpec((16, H), lambda i: (0, 0)),
        pl.BlockSpec((1, H), lambda i: (0, 0)),
        pl.BlockSpec((H, C), lambda i: (0, 0)),
        pl.BlockSpec((1, C), lambda i: (0, 0)),
    ],
    out_specs=pl.BlockSpec((2000, C), lambda i: (i, 0)),
    out_shape=jax.ShapeDtypeStruct((E, C), jnp.float32),
    compiler_params=_TC_PARAMS,
)


_CH = 80
_EPW = E // NW
_NIT = _EPW // _CH
_EPT = E // NS
_NIT_G = _EPT // _CH


@functools.cache
def _sc_kernels():
    mesh = plsc.VectorSubcoreMesh(core_axis_name="c", subcore_axis_name="s",
                                  num_cores=NC, num_subcores=NS)
    sc_params = pltpu.CompilerParams(use_tc_tiling_on_sc=False)

    def make_segsum(D):
        @functools.partial(
            pl.kernel,
            out_type=jax.ShapeDtypeStruct((2 * NP, D), jnp.float32),
            mesh=mesh,
            compiler_params=sc_params,
            scratch_types=[
                pltpu.VMEM((_NIT, _CH), jnp.int32),
                pltpu.VMEM((_CH,), jnp.int32),
                pltpu.VMEM((_CH,), jnp.int32),
                pltpu.VMEM((_CH, D), jnp.float32),
                pltpu.VMEM((_CH, D), jnp.float32),
                pltpu.SemaphoreType.DMA,
                pltpu.SemaphoreType.DMA,
                pltpu.SemaphoreType.DMA,
                pltpu.SemaphoreType.DMA,
                pltpu.SemaphoreType.DMA,
                pltpu.SemaphoreType.DMA,
                pltpu.VMEM_SHARED((NP, D), jnp.float32),
            ],
        )
        def segsum(table, src3, dstd, zeros, out,
                   si_all, di0, di1, rows0, rows1,
                   id0, id1, gs0, gs1, as0, as1, acc):
            c = lax.axis_index("c")
            s = lax.axis_index("s")
            wid = c * NS + s
            base = wid * _EPW
            pltpu.sync_copy(zeros, acc.at[pl.ds(s * RPT, RPT)])
            pltpu.sync_copy(src3.at[wid], si_all)
            plsc.subcore_barrier()

            def diload(i, di, sem):
                pltpu.async_copy(dstd.at[pl.ds(base + i * _CH, _CH)], di, sem)

            def diwait(i, di, sem):
                pltpu.make_async_copy(dstd.at[pl.ds(base + i * _CH, _CH)],
                                      di, sem).wait()

            def gath(i, rows, sem):
                pltpu.async_copy(table.at[si_all.at[i]], rows, sem)

            def gwait(i, rows, sem):
                pltpu.make_async_copy(table.at[si_all.at[i]], rows, sem).wait()

            def scat(di, rows, sem):
                pltpu.async_copy(rows, acc.at[di], sem, add=True)

            def swait(di, rows, sem):
                pltpu.make_async_copy(rows, acc.at[di], sem).wait()

            diload(0, di0, id0)
            diload(1, di1, id1)
            gath(0, rows0, gs0)
            gath(1, rows1, gs1)

            def step2(k, carry):
                i0 = 2 * k
                i1 = i0 + 1
                gwait(i0, rows0, gs0)
                diwait(i0, di0, id0)
                scat(di0, rows0, as0)
                gwait(i1, rows1, gs1)
                diwait(i1, di1, id1)
                scat(di1, rows1, as1)
                swait(di0, rows0, as0)
                gath(i0 + 2, rows0, gs0)
                diload(i0 + 2, di0, id0)
                swait(di1, rows1, as1)

                @pl.when(i1 + 2 < _NIT)
                def _():
                    gath(i1 + 2, rows1, gs1)
                    diload(i1 + 2, di1, id1)

                return carry

            lax.fori_loop(0, (_NIT - 1) // 2, step2, 0)
            gwait(_NIT - 1, rows0, gs0)
            diwait(_NIT - 1, di0, id0)
            scat(di0, rows0, as0)
            swait(di0, rows0, as0)

            plsc.subcore_barrier()
            pltpu.sync_copy(acc.at[pl.ds(s * RPT, RPT)],
                            out.at[pl.ds(c * NP + s * RPT, RPT)])

        return segsum

    @functools.partial(
        pl.kernel,
        out_type=jax.ShapeDtypeStruct((E, H), jnp.float32),
        mesh=mesh,
        compiler_params=sc_params,
        scratch_types=[
            pltpu.VMEM((_NIT_G, _CH), jnp.int32),
            pltpu.VMEM((_CH, H), jnp.float32),
            pltpu.VMEM((_CH, H), jnp.float32),
            pltpu.SemaphoreType.DMA,
            pltpu.SemaphoreType.DMA,
            pltpu.SemaphoreType.DMA,
            pltpu.SemaphoreType.DMA,
        ],
    )
    def gather2(tab, src2, dst2, g_out,
                idx_all, rows0, rows1, sem0, sem1, ss0, ss1):
        c = lax.axis_index("c")
        s = lax.axis_index("s")
        base = s * _EPT

        def run(idx2, lo):
            pltpu.sync_copy(idx2.at[s], idx_all)

            def gath(i, rows, sem):
                pltpu.async_copy(tab.at[idx_all.at[i]], rows, sem)

            def gwait(i, rows, sem):
                pltpu.make_async_copy(tab.at[idx_all.at[i]], rows, sem).wait()

            def dsts(i, rows):
                return (rows.at[:, pl.ds(lo, 64)],
                        g_out.at[pl.ds(base + i * _CH, _CH), pl.ds(lo, 64)])

            def store(i, rows, sem):
                a, b = dsts(i, rows)
                pltpu.async_copy(a, b, sem)

            def swait(i, rows, sem):
                a, b = dsts(i, rows)
                pltpu.make_async_copy(a, b, sem).wait()

            gath(0, rows0, sem0)
            gath(1, rows1, sem1)

            def step2(k, carry):
                i0 = 2 * k
                i1 = i0 + 1
                gwait(i0, rows0, sem0)
                store(i0, rows0, ss0)
                gwait(i1, rows1, sem1)
                store(i1, rows1, ss1)
                swait(i0, rows0, ss0)
                gath(i0 + 2, rows0, sem0)
                swait(i1, rows1, ss1)
                gath(i1 + 2, rows1, sem1)
                return carry

            lax.fori_loop(0, _NIT_G // 2 - 1, step2, 0)
            gwait(_NIT_G - 2, rows0, sem0)
            store(_NIT_G - 2, rows0, ss0)
            gwait(_NIT_G - 1, rows1, sem1)
            store(_NIT_G - 1, rows1, ss1)
            swait(_NIT_G - 2, rows0, ss0)
            swait(_NIT_G - 1, rows1, ss1)

        @pl.when(c == 0)
        def _():
            run(src2, 0)

        @pl.when(c == 1)
        def _():
            run(dst2, 64)

    return make_segsum(DP), make_segsum(H), gather2


def kernel(x, edge_index, edge_attr, W_emb, b_emb, Wl1, bl1, Wr1, br1,
           Wl2, bl2, Wr2, br2, Wh, bh, Wo, bo):
    segsum_dp, segsum_h, gather2 = _sc_kernels()
    src = edge_index[0]
    dst = edge_index[1]
    src3 = src.reshape(NW, _NIT, _CH)
    src2 = src.reshape(NS, _NIT_G, _CH)
    dst2 = dst.reshape(NS, _NIT_G, _CH)

    h0p = _embed(x, W_emb, b_emb.reshape(1, H))

    zeros_dp = jnp.zeros((RPT, DP), jnp.float32)
    agg1 = segsum_dp(h0p, src3, dst, zeros_dp).reshape(2, NP, DP)

    h1 = _layer1(agg1, agg1, h0p, Wl1, Wr1, (bl1 + br1).reshape(1, H))

    zeros_h = jnp.zeros((RPT, H), jnp.float32)
    agg2 = segsum_h(h1, src3, dst, zeros_h).reshape(2, NP, H)

    T = _layer2(agg2, agg2, agg1, agg1, h1, Wl2, Wr2,
                (bl2 + br2).reshape(1, H), Wh[:H], Wh[H:2 * H])

    g = gather2(T, src2, dst2)

    return _classifier(g, edge_attr, Wh[2 * H:], bh.reshape(1, H),
                       Wo, bo.reshape(1, C))


# speedup vs baseline: 1.0527x; 1.0527x over previous
import functools

import jax
import jax.numpy as jnp
from jax import lax
from jax.experimental import pallas as pl
from jax.experimental.pallas import tpu as pltpu
from jax.experimental.pallas import tpu_sc as plsc

N = 10000
E = 320000
H = 128
DP = 144
C = 8

NC, NS = 2, 16
NW = NC * NS
NP = 10240
RPT = NP // NS

_TC_PARAMS = pltpu.CompilerParams(dimension_semantics=("parallel",))


def _embed_body(x_ref, w_ref, b_ref, out_ref):
    h = jnp.dot(x_ref[...], w_ref[...], preferred_element_type=jnp.float32)
    h = h + b_ref[...]
    ones = jnp.ones((h.shape[0], DP - H), jnp.float32)
    out_ref[...] = jnp.concatenate([h, ones], axis=1)


_embed = pl.pallas_call(
    _embed_body,
    grid=(10,),
    in_specs=[
        pl.BlockSpec((1000, H), lambda i: (i, 0)),
        pl.BlockSpec((H, H), lambda i: (0, 0)),
        pl.BlockSpec((1, H), lambda i: (0, 0)),
    ],
    out_specs=pl.BlockSpec((1000, DP), lambda i: (i, 0)),
    out_shape=jax.ShapeDtypeStruct((N, DP), jnp.float32),
    compiler_params=_TC_PARAMS,
)


def _layer1_body(a0_ref, a1_ref, hp_ref, wl_ref, wr_ref, b_ref, out_ref):
    a = a0_ref[...] + a1_ref[...]
    deg = a[:, H:H + 1]
    cinv = 1.0 / jnp.maximum(deg, 1.0)
    mean = a[:, :H] * cinv
    h0 = hp_ref[:, :H]
    out = (jnp.dot(mean, wl_ref[...], preferred_element_type=jnp.float32)
           + jnp.dot(h0, wr_ref[...], preferred_element_type=jnp.float32)
           + b_ref[...])
    out_ref[...] = jnp.maximum(out, 0.0)


_layer1 = pl.pallas_call(
    _layer1_body,
    grid=(10,),
    in_specs=[
        pl.BlockSpec((None, 1000, DP), lambda i: (0, i, 0)),
        pl.BlockSpec((None, 1000, DP), lambda i: (1, i, 0)),
        pl.BlockSpec((1000, DP), lambda i: (i, 0)),
        pl.BlockSpec((H, H), lambda i: (0, 0)),
        pl.BlockSpec((H, H), lambda i: (0, 0)),
        pl.BlockSpec((1, H), lambda i: (0, 0)),
    ],
    out_specs=pl.BlockSpec((1000, H), lambda i: (i, 0)),
    out_shape=jax.ShapeDtypeStruct((N, H), jnp.float32),
    compiler_params=_TC_PARAMS,
)


def _layer2_body(a0_ref, a1_ref, c0_ref, c1_ref, h1_ref, wl_ref, wr_ref,
                 b_ref, whs_ref, whd_ref, t_out):
    a = a0_ref[...] + a1_ref[...]
    deg = c0_ref[:, H:H + 1] + c1_ref[:, H:H + 1]
    cinv = 1.0 / jnp.maximum(deg, 1.0)
    mean = a * cinv
    h2 = (jnp.dot(mean, wl_ref[...], preferred_element_type=jnp.float32)
          + jnp.dot(h1_ref[...], wr_ref[...], preferred_element_type=jnp.float32)
          + b_ref[...])
    h2 = jnp.maximum(h2, 0.0)

    def pack(v):
        u = jax.lax.bitcast_convert_type(v, jnp.int32)
        r = (u + 0x7FFF + ((u >> 16) & 1)) & jnp.int32(-65536)
        hi = r[:, :64]
        lo = (r[:, 64:] >> 16) & 0xFFFF
        return jax.lax.bitcast_convert_type(hi | lo, jnp.float32)

    t_out[...] = jnp.concatenate(
        [pack(jnp.dot(h2, whs_ref[...], preferred_element_type=jnp.float32)),
         pack(jnp.dot(h2, whd_ref[...], preferred_element_type=jnp.float32))],
        axis=1)


_layer2 = pl.pallas_call(
    _layer2_body,
    grid=(10,),
    in_specs=[
        pl.BlockSpec((None, 1000, H), lambda i: (0, i, 0)),
        pl.BlockSpec((None, 1000, H), lambda i: (1, i, 0)),
        pl.BlockSpec((None, 1000, DP), lambda i: (0, i, 0)),
        pl.BlockSpec((None, 1000, DP), lambda i: (1, i, 0)),
        pl.BlockSpec((1000, H), lambda i: (i, 0)),
        pl.BlockSpec((H, H), lambda i: (0, 0)),
        pl.BlockSpec((H, H), lambda i: (0, 0)),
        pl.BlockSpec((1, H), lambda i: (0, 0)),
        pl.BlockSpec((H, H), lambda i: (0, 0)),
        pl.BlockSpec((H, H), lambda i: (0, 0)),
    ],
    out_specs=pl.BlockSpec((1000, H), lambda i: (i, 0)),
    out_shape=jax.ShapeDtypeStruct((N, H), jnp.float32),
    compiler_params=_TC_PARAMS,
)


def _classifier_body(g_ref, ea_ref, whe_ref, bh_ref, wo_ref, bo_ref,
                     out_ref):
    w = jax.lax.bitcast_convert_type(g_ref[...], jnp.int32)
    hi = jax.lax.bitcast_convert_type(w & jnp.int32(-65536), jnp.float32)
    lo = jax.lax.bitcast_convert_type(w << 16, jnp.float32)
    g = jnp.concatenate([hi[:, :64] + hi[:, 64:], lo[:, :64] + lo[:, 64:]],
                        axis=1)
    z = (g
         + jnp.dot(ea_ref[...], whe_ref[...], preferred_element_type=jnp.float32)
         + bh_ref[...])
    z = jnp.maximum(z, 0.0)
    out_ref[...] = jnp.dot(z, wo_ref[...], preferred_element_type=jnp.float32) + bo_ref[...]


_classifier = pl.pallas_call(
    _classifier_body,
    grid=(160,),
    in_specs=[
        pl.BlockSpec((2000, H), lambda i: (i, 0)),
        pl.BlockSpec((2000, 16), lambda i: (i, 0)),
        pl.BlockSpec((16, H), lambda i: (0, 0)),
        pl.BlockSpec((1, H), lambda i: (0, 0)),
        pl.BlockSpec((H, C), lambda i: (0, 0)),
        pl.BlockSpec((1, C), lambda i: (0, 0)),
    ],
    out_specs=pl.BlockSpec((2000, C), lambda i: (i, 0)),
    out_shape=jax.ShapeDtypeStruct((E, C), jnp.float32),
    compiler_params=_TC_PARAMS,
)


_CH = 80
_EPW = E // NW
_NIT = _EPW // _CH
_EPT = E // NS
_NIT_G = _EPT // _CH


@functools.cache
def _sc_kernels():
    mesh = plsc.VectorSubcoreMesh(core_axis_name="c", subcore_axis_name="s",
                                  num_cores=NC, num_subcores=NS)
    sc_params = pltpu.CompilerParams(use_tc_tiling_on_sc=False)

    def make_segsum(D):
        @functools.partial(
            pl.kernel,
            out_type=jax.ShapeDtypeStruct((2 * NP, D), jnp.float32),
            mesh=mesh,
            compiler_params=sc_params,
            scratch_types=[
                pltpu.VMEM((_CH,), jnp.int32),
                pltpu.VMEM((_CH,), jnp.int32),
                pltpu.VMEM((_CH,), jnp.int32),
                pltpu.VMEM((_CH,), jnp.int32),
                pltpu.VMEM((_CH, D), jnp.float32),
                pltpu.VMEM((_CH, D), jnp.float32),
                pltpu.SemaphoreType.DMA,
                pltpu.SemaphoreType.DMA,
                pltpu.SemaphoreType.DMA,
                pltpu.SemaphoreType.DMA,
                pltpu.SemaphoreType.DMA,
                pltpu.SemaphoreType.DMA,
                pltpu.VMEM_SHARED((NP, D), jnp.float32),
            ],
        )
        def segsum(table, srcd, dstd, zeros, out,
                   si0, di0, si1, di1, rows0, rows1,
                   is0, id0, is1, id1, gs0, gs1, acc):
            c = lax.axis_index("c")
            s = lax.axis_index("s")
            wid = c * NS + s
            base = wid * _EPW
            pltpu.sync_copy(zeros, acc.at[pl.ds(s * RPT, RPT)])
            plsc.subcore_barrier()

            def idx_load(i, si, di, ssem, dsem):
                off = base + i * _CH
                pltpu.async_copy(srcd.at[pl.ds(off, _CH)], si, ssem)
                pltpu.async_copy(dstd.at[pl.ds(off, _CH)], di, dsem)

            def idx_wait(i, si, di, ssem, dsem):
                off = base + i * _CH
                pltpu.make_async_copy(srcd.at[pl.ds(off, _CH)], si, ssem).wait()
                pltpu.make_async_copy(dstd.at[pl.ds(off, _CH)], di, dsem).wait()

            def gath(si, rows, sem):
                pltpu.async_copy(table.at[si], rows, sem)

            def gwait(si, rows, sem):
                pltpu.make_async_copy(table.at[si], rows, sem).wait()

            def scat(di, rows):
                pltpu.sync_copy(rows, acc.at[di], add=True)

            idx_load(0, si0, di0, is0, id0)
            idx_load(1, si1, di1, is1, id1)

            def step2(k, carry):
                i0 = 2 * k
                i1 = i0 + 1
                idx_wait(i0, si0, di0, is0, id0)
                gath(si0, rows0, gs0)
                idx_wait(i1, si1, di1, is1, id1)
                gath(si1, rows1, gs1)
                gwait(si0, rows0, gs0)
                scat(di0, rows0)
                idx_load(i0 + 2, si0, di0, is0, id0)
                gwait(si1, rows1, gs1)
                scat(di1, rows1)

                @pl.when(i1 + 2 < _NIT)
                def _():
                    idx_load(i1 + 2, si1, di1, is1, id1)

                return carry

            lax.fori_loop(0, (_NIT - 1) // 2, step2, 0)
            idx_wait(_NIT - 1, si0, di0, is0, id0)
            gath(si0, rows0, gs0)
            gwait(si0, rows0, gs0)
            scat(di0, rows0)

            plsc.subcore_barrier()
            pltpu.sync_copy(acc.at[pl.ds(s * RPT, RPT)],
                            out.at[pl.ds(c * NP + s * RPT, RPT)])

        return segsum

    @functools.partial(
        pl.kernel,
        out_type=jax.ShapeDtypeStruct((E, H), jnp.float32),
        mesh=mesh,
        compiler_params=sc_params,
        scratch_types=[
            pltpu.VMEM((_NIT_G, _CH), jnp.int32),
            pltpu.VMEM((_CH, H), jnp.float32),
            pltpu.VMEM((_CH, H), jnp.float32),
            pltpu.SemaphoreType.DMA,
            pltpu.SemaphoreType.DMA,
        ],
    )
    def gather2(tab, src2, dst2, g_out, idx_all, rows0, rows1, sem0, sem1):
        c = lax.axis_index("c")
        s = lax.axis_index("s")
        base = s * _EPT

        def run(idx2, lo):
            pltpu.sync_copy(idx2.at[s], idx_all)

            def gath(i, rows, sem):
                pltpu.async_copy(tab.at[idx_all.at[i]], rows, sem)

            def wait(i, rows, sem):
                pltpu.make_async_copy(tab.at[idx_all.at[i]], rows, sem).wait()

            def store(i, rows):
                pltpu.sync_copy(
                    rows.at[:, pl.ds(lo, 64)],
                    g_out.at[pl.ds(base + i * _CH, _CH), pl.ds(lo, 64)])

            gath(0, rows0, sem0)

            def step2(k, carry):
                i0 = 2 * k
                gath(i0 + 1, rows1, sem1)
                wait(i0, rows0, sem0)
                store(i0, rows0)
                gath(i0 + 2, rows0, sem0)
                wait(i0 + 1, rows1, sem1)
                store(i0 + 1, rows1)
                return carry

            lax.fori_loop(0, _NIT_G // 2 - 1, step2, 0)
            gath(_NIT_G - 1, rows1, sem1)
            wait(_NIT_G - 2, rows0, sem0)
            store(_NIT_G - 2, rows0)
            wait(_NIT_G - 1, rows1, sem1)
            store(_NIT_G - 1, rows1)

        @pl.when(c == 0)
        def _():
            run(src2, 0)

        @pl.when(c == 1)
        def _():
            run(dst2, 64)

    return make_segsum(DP), make_segsum(H), gather2


def kernel(x, edge_index, edge_attr, W_emb, b_emb, Wl1, bl1, Wr1, br1,
           Wl2, bl2, Wr2, br2, Wh, bh, Wo, bo):
    segsum_dp, segsum_h, gather2 = _sc_kernels()
    src = edge_index[0]
    dst = edge_index[1]
    src2 = src.reshape(NS, _NIT_G, _CH)
    dst2 = dst.reshape(NS, _NIT_G, _CH)

    h0p = _embed(x, W_emb, b_emb.reshape(1, H))

    zeros_dp = jnp.zeros((RPT, DP), jnp.float32)
    agg1 = segsum_dp(h0p, src, dst, zeros_dp).reshape(2, NP, DP)

    h1 = _layer1(agg1, agg1, h0p, Wl1, Wr1, (bl1 + br1).reshape(1, H))

    zeros_h = jnp.zeros((RPT, H), jnp.float32)
    agg2 = segsum_h(h1, src, dst, zeros_h).reshape(2, NP, H)

    T = _layer2(agg2, agg2, agg1, agg1, h1, Wl2, Wr2,
                (bl2 + br2).reshape(1, H), Wh[:H], Wh[H:2 * H])

    g = gather2(T, src2, dst2)

    return _classifier(g, edge_attr, Wh[2 * H:], bh.reshape(1, H),
                       Wo, bo.reshape(1, C))


# speedup vs baseline: 1.1415x; 1.0844x over previous
import functools

import jax
import jax.numpy as jnp
from jax import lax
from jax.experimental import pallas as pl
from jax.experimental.pallas import tpu as pltpu
from jax.experimental.pallas import tpu_sc as plsc

N = 10000
E = 320000
H = 128
DP = 144
C = 8

NC, NS = 2, 16
NW = NC * NS
NP = 10240
RPT = NP // NS

_TC_PARAMS = pltpu.CompilerParams(dimension_semantics=("parallel",))


def _embed_body(x_ref, w_ref, b_ref, out_ref):
    h = jnp.dot(x_ref[...], w_ref[...], preferred_element_type=jnp.float32)
    h = h + b_ref[...]
    ones = jnp.ones((h.shape[0], DP - H), jnp.float32)
    out_ref[...] = jnp.concatenate([h, ones], axis=1)


_embed = pl.pallas_call(
    _embed_body,
    grid=(10,),
    in_specs=[
        pl.BlockSpec((1000, H), lambda i: (i, 0)),
        pl.BlockSpec((H, H), lambda i: (0, 0)),
        pl.BlockSpec((1, H), lambda i: (0, 0)),
    ],
    out_specs=pl.BlockSpec((1000, DP), lambda i: (i, 0)),
    out_shape=jax.ShapeDtypeStruct((N, DP), jnp.float32),
    compiler_params=_TC_PARAMS,
)


def _layer1_body(a0_ref, a1_ref, hp_ref, wl_ref, wr_ref, b_ref, out_ref):
    a = a0_ref[...] + a1_ref[...]
    deg = a[:, H:H + 1]
    cinv = 1.0 / jnp.maximum(deg, 1.0)
    mean = a[:, :H] * cinv
    h0 = hp_ref[:, :H]
    out = (jnp.dot(mean, wl_ref[...], preferred_element_type=jnp.float32)
           + jnp.dot(h0, wr_ref[...], preferred_element_type=jnp.float32)
           + b_ref[...])
    out_ref[...] = jnp.maximum(out, 0.0)


_layer1 = pl.pallas_call(
    _layer1_body,
    grid=(10,),
    in_specs=[
        pl.BlockSpec((None, 1000, DP), lambda i: (0, i, 0)),
        pl.BlockSpec((None, 1000, DP), lambda i: (1, i, 0)),
        pl.BlockSpec((1000, DP), lambda i: (i, 0)),
        pl.BlockSpec((H, H), lambda i: (0, 0)),
        pl.BlockSpec((H, H), lambda i: (0, 0)),
        pl.BlockSpec((1, H), lambda i: (0, 0)),
    ],
    out_specs=pl.BlockSpec((1000, H), lambda i: (i, 0)),
    out_shape=jax.ShapeDtypeStruct((N, H), jnp.float32),
    compiler_params=_TC_PARAMS,
)


def _layer2_body(a0_ref, a1_ref, c0_ref, c1_ref, h1_ref, wl_ref, wr_ref,
                 b_ref, whs_ref, whd_ref, t_out):
    a = a0_ref[...] + a1_ref[...]
    deg = c0_ref[:, H:H + 1] + c1_ref[:, H:H + 1]
    cinv = 1.0 / jnp.maximum(deg, 1.0)
    mean = a * cinv
    h2 = (jnp.dot(mean, wl_ref[...], preferred_element_type=jnp.float32)
          + jnp.dot(h1_ref[...], wr_ref[...], preferred_element_type=jnp.float32)
          + b_ref[...])
    h2 = jnp.maximum(h2, 0.0)

    def pack(v):
        u = jax.lax.bitcast_convert_type(v, jnp.int32)
        r = (u + 0x7FFF + ((u >> 16) & 1)) & jnp.int32(-65536)
        hi = r[:, :64]
        lo = (r[:, 64:] >> 16) & 0xFFFF
        return jax.lax.bitcast_convert_type(hi | lo, jnp.float32)

    t_out[...] = jnp.concatenate(
        [pack(jnp.dot(h2, whs_ref[...], preferred_element_type=jnp.float32)),
         pack(jnp.dot(h2, whd_ref[...], preferred_element_type=jnp.float32))],
        axis=1)


_layer2 = pl.pallas_call(
    _layer2_body,
    grid=(10,),
    in_specs=[
        pl.BlockSpec((None, 1000, H), lambda i: (0, i, 0)),
        pl.BlockSpec((None, 1000, H), lambda i: (1, i, 0)),
        pl.BlockSpec((None, 1000, DP), lambda i: (0, i, 0)),
        pl.BlockSpec((None, 1000, DP), lambda i: (1, i, 0)),
        pl.BlockSpec((1000, H), lambda i: (i, 0)),
        pl.BlockSpec((H, H), lambda i: (0, 0)),
        pl.BlockSpec((H, H), lambda i: (0, 0)),
        pl.BlockSpec((1, H), lambda i: (0, 0)),
        pl.BlockSpec((H, H), lambda i: (0, 0)),
        pl.BlockSpec((H, H), lambda i: (0, 0)),
    ],
    out_specs=pl.BlockSpec((1000, H), lambda i: (i, 0)),
    out_shape=jax.ShapeDtypeStruct((N, H), jnp.float32),
    compiler_params=_TC_PARAMS,
)


def _classifier_body(g_ref, ea_ref, whe_ref, bh_ref, wo_ref, bo_ref,
                     out_ref):
    w = jax.lax.bitcast_convert_type(g_ref[...], jnp.int32)
    hi = jax.lax.bitcast_convert_type(w & jnp.int32(-65536), jnp.float32)
    lo = jax.lax.bitcast_convert_type(w << 16, jnp.float32)
    g = jnp.concatenate([hi[:, :64] + hi[:, 64:], lo[:, :64] + lo[:, 64:]],
                        axis=1)
    z = (g
         + jnp.dot(ea_ref[...], whe_ref[...], preferred_element_type=jnp.float32)
         + bh_ref[...])
    z = jnp.maximum(z, 0.0)
    out_ref[...] = jnp.dot(z, wo_ref[...], preferred_element_type=jnp.float32) + bo_ref[...]


_classifier = pl.pallas_call(
    _classifier_body,
    grid=(40,),
    in_specs=[
        pl.BlockSpec((8000, H), lambda i: (i, 0)),
        pl.BlockSpec((8000, 16), lambda i: (i, 0)),
        pl.BlockSpec((16, H), lambda i: (0, 0)),
        pl.BlockSpec((1, H), lambda i: (0, 0)),
        pl.BlockSpec((H, C), lambda i: (0, 0)),
        pl.BlockSpec((1, C), lambda i: (0, 0)),
    ],
    out_specs=pl.BlockSpec((8000, C), lambda i: (i, 0)),
    out_shape=jax.ShapeDtypeStruct((E, C), jnp.float32),
    compiler_params=_TC_PARAMS,
)


_CH = 80
_EPW = E // NW
_NIT = _EPW // _CH
_EPT = E // NS
_NIT_G = _EPT // _CH


@functools.cache
def _sc_kernels():
    mesh = plsc.VectorSubcoreMesh(core_axis_name="c", subcore_axis_name="s",
                                  num_cores=NC, num_subcores=NS)
    sc_params = pltpu.CompilerParams(use_tc_tiling_on_sc=False)

    def make_segsum(D):
        @functools.partial(
            pl.kernel,
            out_type=jax.ShapeDtypeStruct((2 * NP, D), jnp.float32),
            mesh=mesh,
            compiler_params=sc_params,
            scratch_types=[
                pltpu.VMEM((_CH,), jnp.int32),
                pltpu.VMEM((_CH,), jnp.int32),
                pltpu.VMEM((_CH,), jnp.int32),
                pltpu.VMEM((_CH,), jnp.int32),
                pltpu.VMEM((_CH, D), jnp.float32),
                pltpu.VMEM((_CH, D), jnp.float32),
                pltpu.SemaphoreType.DMA,
                pltpu.SemaphoreType.DMA,
                pltpu.SemaphoreType.DMA,
                pltpu.SemaphoreType.DMA,
                pltpu.SemaphoreType.DMA,
                pltpu.SemaphoreType.DMA,
                pltpu.VMEM_SHARED((NP, D), jnp.float32),
            ],
        )
        def segsum(table, srcd, dstd, zeros, out,
                   si0, di0, si1, di1, rows0, rows1,
                   is0, id0, is1, id1, gs0, gs1, acc):
            c = lax.axis_index("c")
            s = lax.axis_index("s")
            wid = c * NS + s
            base = wid * _EPW
            pltpu.sync_copy(zeros, acc.at[pl.ds(s * RPT, RPT)])
            plsc.subcore_barrier()

            def idx_load(i, si, di, ssem, dsem):
                off = base + i * _CH
                pltpu.async_copy(srcd.at[pl.ds(off, _CH)], si, ssem)
                pltpu.async_copy(dstd.at[pl.ds(off, _CH)], di, dsem)

            def idx_wait(i, si, di, ssem, dsem):
                off = base + i * _CH
                pltpu.make_async_copy(srcd.at[pl.ds(off, _CH)], si, ssem).wait()
                pltpu.make_async_copy(dstd.at[pl.ds(off, _CH)], di, dsem).wait()

            def gath(si, rows, sem):
                pltpu.async_copy(table.at[si], rows, sem)

            def gwait(si, rows, sem):
                pltpu.make_async_copy(table.at[si], rows, sem).wait()

            def scat(di, rows):
                pltpu.sync_copy(rows, acc.at[di], add=True)

            idx_load(0, si0, di0, is0, id0)
            idx_load(1, si1, di1, is1, id1)

            def step2(k, carry):
                i0 = 2 * k
                i1 = i0 + 1
                idx_wait(i0, si0, di0, is0, id0)
                gath(si0, rows0, gs0)
                idx_wait(i1, si1, di1, is1, id1)
                gath(si1, rows1, gs1)
                gwait(si0, rows0, gs0)
                scat(di0, rows0)
                idx_load(i0 + 2, si0, di0, is0, id0)
                gwait(si1, rows1, gs1)
                scat(di1, rows1)

                @pl.when(i1 + 2 < _NIT)
                def _():
                    idx_load(i1 + 2, si1, di1, is1, id1)

                return carry

            lax.fori_loop(0, (_NIT - 1) // 2, step2, 0)
            idx_wait(_NIT - 1, si0, di0, is0, id0)
            gath(si0, rows0, gs0)
            gwait(si0, rows0, gs0)
            scat(di0, rows0)

            plsc.subcore_barrier()
            pltpu.sync_copy(acc.at[pl.ds(s * RPT, RPT)],
                            out.at[pl.ds(c * NP + s * RPT, RPT)])

        return segsum

    @functools.partial(
        pl.kernel,
        out_type=jax.ShapeDtypeStruct((E, H), jnp.float32),
        mesh=mesh,
        compiler_params=sc_params,
        scratch_types=[
            pltpu.VMEM((_NIT_G, _CH), jnp.int32),
            pltpu.VMEM((_CH, H), jnp.float32),
            pltpu.VMEM((_CH, H), jnp.float32),
            pltpu.SemaphoreType.DMA,
            pltpu.SemaphoreType.DMA,
        ],
    )
    def gather2(tab, src2, dst2, g_out, idx_all, rows0, rows1, sem0, sem1):
        c = lax.axis_index("c")
        s = lax.axis_index("s")
        base = s * _EPT

        def run(idx2, lo):
            pltpu.sync_copy(idx2.at[s], idx_all)

            def gath(i, rows, sem):
                pltpu.async_copy(tab.at[idx_all.at[i]], rows, sem)

            def wait(i, rows, sem):
                pltpu.make_async_copy(tab.at[idx_all.at[i]], rows, sem).wait()

            def store(i, rows):
                pltpu.sync_copy(
                    rows.at[:, pl.ds(lo, 64)],
                    g_out.at[pl.ds(base + i * _CH, _CH), pl.ds(lo, 64)])

            gath(0, rows0, sem0)

            def step2(k, carry):
                i0 = 2 * k
                gath(i0 + 1, rows1, sem1)
                wait(i0, rows0, sem0)
                store(i0, rows0)
                gath(i0 + 2, rows0, sem0)
                wait(i0 + 1, rows1, sem1)
                store(i0 + 1, rows1)
                return carry

            lax.fori_loop(0, _NIT_G // 2 - 1, step2, 0)
            gath(_NIT_G - 1, rows1, sem1)
            wait(_NIT_G - 2, rows0, sem0)
            store(_NIT_G - 2, rows0)
            wait(_NIT_G - 1, rows1, sem1)
            store(_NIT_G - 1, rows1)

        @pl.when(c == 0)
        def _():
            run(src2, 0)

        @pl.when(c == 1)
        def _():
            run(dst2, 64)

    return make_segsum(DP), make_segsum(H), gather2


def kernel(x, edge_index, edge_attr, W_emb, b_emb, Wl1, bl1, Wr1, br1,
           Wl2, bl2, Wr2, br2, Wh, bh, Wo, bo):
    segsum_dp, segsum_h, gather2 = _sc_kernels()
    src = edge_index[0]
    dst = edge_index[1]
    src2 = src.reshape(NS, _NIT_G, _CH)
    dst2 = dst.reshape(NS, _NIT_G, _CH)

    h0p = _embed(x, W_emb, b_emb.reshape(1, H))

    zeros_dp = jnp.zeros((RPT, DP), jnp.float32)
    agg1 = segsum_dp(h0p, src, dst, zeros_dp).reshape(2, NP, DP)

    h1 = _layer1(agg1, agg1, h0p, Wl1, Wr1, (bl1 + br1).reshape(1, H))

    zeros_h = jnp.zeros((RPT, H), jnp.float32)
    agg2 = segsum_h(h1, src, dst, zeros_h).reshape(2, NP, H)

    T = _layer2(agg2, agg2, agg1, agg1, h1, Wl2, Wr2,
                (bl2 + br2).reshape(1, H), Wh[:H], Wh[H:2 * H])

    g = gather2(T, src2, dst2)

    return _classifier(g, edge_attr, Wh[2 * H:], bh.reshape(1, H),
                       Wo, bo.reshape(1, C))
